# baseline (speedup 1.0000x reference)
import functools

import jax
import jax.numpy as jnp
from jax import lax
from jax.experimental import pallas as pl
from jax.experimental.pallas import tpu as pltpu
from jax.experimental.pallas import tpu_sc as plsc

NQ = 1024
ND = 16
NK = 100000
KT = 2048
N_TILES = (NK + KT - 1) // KT


_NCH = KT // 128
_QB = NQ // 8


def _argmin_body(q_ref, k_ref, idx_out, scores_s, qsq_s, val_s, chk_s):
    g = pl.program_id(0)
    q = q_ref[...]
    k = k_ref[...]
    k_sq = jnp.sum(k * k, axis=1)
    q2 = q * (-2.0)

    @pl.when(g == 0)
    def _():
        qsq_s[...] = jnp.sum(q * q, axis=1, keepdims=True)
        val_s[...] = jnp.full((NQ, 128), jnp.inf, jnp.float32)
        chk_s[...] = jnp.zeros((NQ, 128), jnp.int32)

    _SUB = 8
    _SW = KT // _SUB
    for j in range(_SUB):
        sl = slice(j * _SW, (j + 1) * _SW)
        scores_s[:, sl] = lax.dot_general(
            q2, k[sl, :], dimension_numbers=(((1,), (1,)), ((), ())),
            preferred_element_type=jnp.float32)

    qsq = jnp.broadcast_to(qsq_s[...], (NQ, 128))

    def _reduce(masked):
        av = val_s[...]
        ac = chk_s[...]
        iota = lax.broadcasted_iota(jnp.int32, (NQ, 128), 1)
        for ch in range(_NCH):
            s = scores_s[:, ch * 128:(ch + 1) * 128]
            d = (qsq + s) + k_sq[None, ch * 128:(ch + 1) * 128]
            if masked:
                d = jnp.where((g * _NCH + ch) * 128 + iota < NK, d, jnp.inf)
            m = d < av
            av = jnp.where(m, d, av)
            ac = jnp.where(m, g * _NCH + ch, ac)
        val_s[...] = av
        chk_s[...] = ac

    @pl.when(g < N_TILES - 1)
    def _():
        _reduce(masked=False)

    @pl.when(g == N_TILES - 1)
    def _():
        _reduce(masked=True)
        av = val_s[...]
        mn = jnp.min(av, axis=1, keepdims=True)
        col = chk_s[...] * 128 + lax.broadcasted_iota(jnp.int32, (NQ, 128), 1)
        cand = jnp.where(av == mn, col, jnp.int32(2**31 - 1))
        idx_out[...] = jnp.min(cand, axis=1)


_argmin_call = pl.pallas_call(
    _argmin_body,
    grid=(N_TILES,),
    in_specs=[
        pl.BlockSpec((NQ, ND), lambda g: (0, 0)),
        pl.BlockSpec((KT, ND), lambda g: (g, 0)),
    ],
    out_specs=pl.BlockSpec((NQ,), lambda g: (0,)),
    out_shape=jax.ShapeDtypeStruct((NQ,), jnp.int32),
    scratch_shapes=[
        pltpu.VMEM((NQ, KT), jnp.float32),
        pltpu.VMEM((NQ, 1), jnp.float32),
        pltpu.VMEM((NQ, 128), jnp.float32),
        pltpu.VMEM((NQ, 128), jnp.int32),
    ],
)


_SC_CORES = 2
_SC_SUBCORES = 16
_NW = _SC_CORES * _SC_SUBCORES
_BPW = NQ // _NW
_GROUP = 128 // ND
_NROWS = NK // _GROUP


@functools.cache
def _make_gather_groups():
    @functools.partial(
        pl.kernel,
        mesh=plsc.VectorSubcoreMesh(core_axis_name="c", subcore_axis_name="s"),
        out_type=jax.ShapeDtypeStruct((NQ, 128), jnp.float32),
        scratch_types=[
            pltpu.VMEM((_BPW,), jnp.int32),
            pltpu.VMEM((_BPW,), jnp.int32),
            pltpu.VMEM((_BPW, 128), jnp.float32),
            pltpu.SemaphoreType.DMA,
        ],
    )
    def _gather_groups(table_hbm, idx_hbm, out_hbm, idx_v, idx8_v, rows_v, sem):
        wid = lax.axis_index("s") * _SC_CORES + lax.axis_index("c")
        base = wid * _BPW
        pltpu.sync_copy(idx_hbm.at[pl.ds(base, _BPW)], idx_v)
        for c in range(_BPW // 16):
            v = idx_v[pl.ds(c * 16, 16)]
            idx8_v[pl.ds(c * 16, 16)] = lax.shift_right_logical(v, 3)
        pltpu.async_copy(table_hbm.at[idx8_v], rows_v, sem).wait()
        pltpu.sync_copy(rows_v, out_hbm.at[pl.ds(base, _BPW)])

    return _gather_groups


def _extract_body(rows_ref, idx_ref, out_ref):
    rem = idx_ref[...] & (_GROUP - 1)
    acc = jnp.zeros((NQ, ND), jnp.float32)
    for j in range(_GROUP):
        acc = jnp.where(rem == j, rows_ref[:, j * ND:(j + 1) * ND], acc)
    out_ref[...] = acc


_extract_call = pl.pallas_call(
    _extract_body,
    out_shape=jax.ShapeDtypeStruct((NQ, ND), jnp.float32),
)


def kernel(queries, keys):
    best_idx = _argmin_call(queries, keys)
    rows8 = _make_gather_groups()(keys.reshape(_NROWS, 128), best_idx)
    best_vecs = _extract_call(rows8, best_idx[:, None])
    return best_idx, best_vecs


# speedup vs baseline: 1.2523x; 1.2523x over previous
import functools

import jax
import jax.numpy as jnp
from jax import lax
from jax.experimental import pallas as pl
from jax.experimental.pallas import tpu as pltpu
from jax.experimental.pallas import tpu_sc as plsc

NQ = 1024
ND = 16
NK = 100000
KT = 2048
KPAD = 102400
N_TILES = KPAD // KT


_NCH = KT // 128
_QB = NQ // 8


def _argmin_body(q_ref, k_ref, idx_out, scores_s, qsq_s, val_s, chk_s):
    g = pl.program_id(0)
    q = q_ref[...]
    k = k_ref[...]
    k_sq = jnp.sum(k * k, axis=1)
    q2 = q * (-2.0)

    @pl.when(g == 0)
    def _():
        qsq_s[...] = jnp.sum(q * q, axis=1, keepdims=True)
        val_s[...] = jnp.full((NQ, 128), jnp.inf, jnp.float32)
        chk_s[...] = jnp.zeros((NQ, 128), jnp.int32)

    _SUB = 8
    _SW = KT // _SUB
    for j in range(_SUB):
        sl = slice(j * _SW, (j + 1) * _SW)
        scores_s[:, sl] = lax.dot_general(
            q2, k[sl, :], dimension_numbers=(((1,), (1,)), ((), ())),
            preferred_element_type=jnp.float32)

    qsq = jnp.broadcast_to(qsq_s[...], (NQ, 128))
    av = val_s[...]
    ac = chk_s[...]
    for ch in range(_NCH):
        s = scores_s[:, ch * 128:(ch + 1) * 128]
        d = (qsq + s) + k_sq[None, ch * 128:(ch + 1) * 128]
        m = d < av
        av = jnp.where(m, d, av)
        ac = jnp.where(m, g * _NCH + ch, ac)
    val_s[...] = av
    chk_s[...] = ac

    @pl.when(g == N_TILES - 1)
    def _():
        av = val_s[...]
        mn = jnp.min(av, axis=1, keepdims=True)
        col = chk_s[...] * 128 + lax.broadcasted_iota(jnp.int32, (NQ, 128), 1)
        cand = jnp.where(av == mn, col, jnp.int32(2**31 - 1))
        idx_out[...] = jnp.min(cand, axis=1)


_argmin_call = pl.pallas_call(
    _argmin_body,
    grid=(N_TILES,),
    in_specs=[
        pl.BlockSpec((NQ, ND), lambda g: (0, 0)),
        pl.BlockSpec((KT, ND), lambda g: (g, 0)),
    ],
    out_specs=pl.BlockSpec((NQ,), lambda g: (0,)),
    out_shape=jax.ShapeDtypeStruct((NQ,), jnp.int32),
    scratch_shapes=[
        pltpu.VMEM((NQ, KT), jnp.float32),
        pltpu.VMEM((NQ, 1), jnp.float32),
        pltpu.VMEM((NQ, 128), jnp.float32),
        pltpu.VMEM((NQ, 128), jnp.int32),
    ],
)


_SC_CORES = 2
_SC_SUBCORES = 16
_NW = _SC_CORES * _SC_SUBCORES
_BPW = NQ // _NW
_GROUP = 128 // ND
_NROWS = NK // _GROUP


@functools.cache
def _make_gather_groups():
    @functools.partial(
        pl.kernel,
        mesh=plsc.VectorSubcoreMesh(core_axis_name="c", subcore_axis_name="s"),
        out_type=jax.ShapeDtypeStruct((NQ, 128), jnp.float32),
        scratch_types=[
            pltpu.VMEM((_BPW,), jnp.int32),
            pltpu.VMEM((_BPW,), jnp.int32),
            pltpu.VMEM((_BPW, 128), jnp.float32),
            pltpu.SemaphoreType.DMA,
        ],
    )
    def _gather_groups(table_hbm, idx_hbm, out_hbm, idx_v, idx8_v, rows_v, sem):
        wid = lax.axis_index("s") * _SC_CORES + lax.axis_index("c")
        base = wid * _BPW
        pltpu.sync_copy(idx_hbm.at[pl.ds(base, _BPW)], idx_v)
        for c in range(_BPW // 16):
            v = idx_v[pl.ds(c * 16, 16)]
            idx8_v[pl.ds(c * 16, 16)] = lax.shift_right_logical(v, 3)
        pltpu.async_copy(table_hbm.at[idx8_v], rows_v, sem).wait()
        pltpu.sync_copy(rows_v, out_hbm.at[pl.ds(base, _BPW)])

    return _gather_groups


def _extract_body(rows_ref, idx_ref, out_ref):
    rem = idx_ref[...] & (_GROUP - 1)
    acc = jnp.zeros((NQ, ND), jnp.float32)
    for j in range(_GROUP):
        acc = jnp.where(rem == j, rows_ref[:, j * ND:(j + 1) * ND], acc)
    out_ref[...] = acc


_extract_call = pl.pallas_call(
    _extract_body,
    out_shape=jax.ShapeDtypeStruct((NQ, ND), jnp.float32),
)


def kernel(queries, keys):
    keys_pad = jnp.pad(keys, ((0, KPAD - NK), (0, 0)), constant_values=1e18)
    best_idx = _argmin_call(queries, keys_pad)
    rows8 = _make_gather_groups()(keys.reshape(_NROWS, 128), best_idx)
    best_vecs = _extract_call(rows8, best_idx[:, None])
    return best_idx, best_vecs


# speedup vs baseline: 1.5558x; 1.2424x over previous
import functools

import jax
import jax.numpy as jnp
from jax import lax
from jax.experimental import pallas as pl
from jax.experimental.pallas import tpu as pltpu
from jax.experimental.pallas import tpu_sc as plsc

NQ = 1024
ND = 16
NK = 100000
KT = 2048
KPAD = 102400
N_TILES = KPAD // KT


_NCH = KT // 128
_QB = NQ // 8


def _argmin_body(q_ref, kt_ref, idx_out, scores_s, qsq_s, val_s, chk_s):
    g = pl.program_id(0)
    q = q_ref[...]
    kt = kt_ref[...]
    q2 = q * (-2.0)

    _SUB = 8
    _SW = KT // _SUB
    _SCH = _SW // 128

    def _dot(j):
        sl = slice(j * _SW, (j + 1) * _SW)
        scores_s[:, sl] = lax.dot_general(
            q2, kt[:, sl], dimension_numbers=(((1,), (0,)), ((), ())),
            preferred_element_type=jnp.float32)

    _dot(0)
    k_sq = jnp.sum(kt * kt, axis=0)

    @pl.when(g == 0)
    def _():
        qsq_s[...] = jnp.sum(q * q, axis=1, keepdims=True)
        val_s[...] = jnp.full((NQ, 128), jnp.inf, jnp.float32)
        chk_s[...] = jnp.zeros((NQ, 128), jnp.int32)

    qsq = jnp.broadcast_to(qsq_s[...], (NQ, 128))
    av = val_s[...]
    ac = chk_s[...]

    def _reduce(j, av, ac):
        for c in range(_SCH):
            ch = j * _SCH + c
            s = scores_s[:, ch * 128:(ch + 1) * 128]
            d = (qsq + s) + k_sq[None, ch * 128:(ch + 1) * 128]
            m = d < av
            av = jnp.where(m, d, av)
            ac = jnp.where(m, g * _NCH + ch, ac)
        return av, ac

    for j in range(1, _SUB):
        _dot(j)
        av, ac = _reduce(j - 1, av, ac)
    av, ac = _reduce(_SUB - 1, av, ac)
    val_s[...] = av
    chk_s[...] = ac

    @pl.when(g == N_TILES - 1)
    def _():
        av = val_s[...]
        mn = jnp.min(av, axis=1, keepdims=True)
        col = chk_s[...] * 128 + lax.broadcasted_iota(jnp.int32, (NQ, 128), 1)
        cand = jnp.where(av == mn, col, jnp.int32(2**31 - 1))
        idx_out[...] = jnp.min(cand, axis=1)


_argmin_call = pl.pallas_call(
    _argmin_body,
    grid=(N_TILES,),
    in_specs=[
        pl.BlockSpec((NQ, ND), lambda g: (0, 0)),
        pl.BlockSpec((ND, KT), lambda g: (0, g)),
    ],
    out_specs=pl.BlockSpec((NQ,), lambda g: (0,)),
    out_shape=jax.ShapeDtypeStruct((NQ,), jnp.int32),
    scratch_shapes=[
        pltpu.VMEM((NQ, KT), jnp.float32),
        pltpu.VMEM((NQ, 1), jnp.float32),
        pltpu.VMEM((NQ, 128), jnp.float32),
        pltpu.VMEM((NQ, 128), jnp.int32),
    ],
)


_SC_CORES = 2
_SC_SUBCORES = 16
_NW = _SC_CORES * _SC_SUBCORES
_BPW = NQ // _NW
_GROUP = 128 // ND
_NROWS = NK // _GROUP


@functools.cache
def _make_gather_groups():
    @functools.partial(
        pl.kernel,
        mesh=plsc.VectorSubcoreMesh(core_axis_name="c", subcore_axis_name="s"),
        out_type=jax.ShapeDtypeStruct((NQ, 128), jnp.float32),
        scratch_types=[
            pltpu.VMEM((_BPW,), jnp.int32),
            pltpu.VMEM((_BPW,), jnp.int32),
            pltpu.VMEM((_BPW, 128), jnp.float32),
            pltpu.SemaphoreType.DMA,
        ],
    )
    def _gather_groups(table_hbm, idx_hbm, out_hbm, idx_v, idx8_v, rows_v, sem):
        wid = lax.axis_index("s") * _SC_CORES + lax.axis_index("c")
        base = wid * _BPW
        pltpu.sync_copy(idx_hbm.at[pl.ds(base, _BPW)], idx_v)
        for c in range(_BPW // 16):
            v = idx_v[pl.ds(c * 16, 16)]
            idx8_v[pl.ds(c * 16, 16)] = lax.shift_right_logical(v, 3)
        pltpu.async_copy(table_hbm.at[idx8_v], rows_v, sem).wait()
        pltpu.sync_copy(rows_v, out_hbm.at[pl.ds(base, _BPW)])

    return _gather_groups


def _extract_body(rows_ref, idx_ref, out_ref):
    rem = idx_ref[...] & (_GROUP - 1)
    acc = jnp.zeros((NQ, ND), jnp.float32)
    for j in range(_GROUP):
        acc = jnp.where(rem == j, rows_ref[:, j * ND:(j + 1) * ND], acc)
    out_ref[...] = acc


_extract_call = pl.pallas_call(
    _extract_body,
    out_shape=jax.ShapeDtypeStruct((NQ, ND), jnp.float32),
)


def kernel(queries, keys):
    keys_pad = jnp.pad(keys, ((0, KPAD - NK), (0, 0)), constant_values=1e18)
    best_idx = _argmin_call(queries, keys_pad.T)
    rows8 = _make_gather_groups()(keys.reshape(_NROWS, 128), best_idx)
    best_vecs = _extract_call(rows8, best_idx[:, None])
    return best_idx, best_vecs
